# baseline (speedup 1.0000x reference)
import functools

import jax
import jax.numpy as jnp
from jax import lax
from jax.experimental import pallas as pl
from jax.experimental.pallas import tpu as pltpu
from jax.experimental.pallas import tpu_sc as plsc

NC = 2
NS = 16
NW = NC * NS
LANES = 16

_GATHER_DNUMS = lax.GatherDimensionNumbers(
    offset_dims=(), collapsed_slice_dims=(0,), start_index_map=(0,))


def _rotate(vec, pattern):
    return lax.gather(vec, pattern[:, None], _GATHER_DNUMS, (1,),
                      mode=lax.GatherScatterMode.PROMISE_IN_BOUNDS)


@functools.partial(jax.jit, static_argnames=("V", "D"))
def _retile(embT, tail, V, D):
    P = 2 * D
    W = 4 * D
    NBLK = V // W
    TAIL = V - NBLK * W
    BASE = NBLK // NW
    mesh = plsc.VectorSubcoreMesh(core_axis_name="c", subcore_axis_name="s")

    @functools.partial(
        pl.kernel,
        mesh=mesh,
        out_type=jax.ShapeDtypeStruct((V // 2, P), jnp.float32),
        scratch_types=[
            pltpu.VMEM((D, W), jnp.float32),
            pltpu.VMEM((D, W), jnp.float32),
            pltpu.VMEM((W // 2, P), jnp.float32),
            pltpu.VMEM((W // 2, P), jnp.float32),
            pltpu.SemaphoreType.DMA,
            pltpu.SemaphoreType.DMA,
            pltpu.SemaphoreType.DMA,
            pltpu.SemaphoreType.DMA,
        ],
        compiler_params=pltpu.CompilerParams(
            use_tc_tiling_on_sc=True, needs_layout_passes=False),
    )
    def body(embT_hbm, tail_hbm, out_hbm, in0, in1, ot0, ot1, is0, is1, os0, os1):
        wid = lax.axis_index("s") * NC + lax.axis_index("c")
        inb = (in0, in1)
        otb = (ot0, ot1)
        isem = (is0, is1)
        osem = (os0, os1)
        nblk = BASE + jnp.where(wid < NBLK - BASE * NW, 1, 0)
        iota = jnp.arange(LANES, dtype=jnp.int32)
        rot = [(iota + r) % LANES for r in range(LANES)]

        def load(k, t):
            b = wid + k * NW
            return pltpu.make_async_copy(
                embT_hbm.at[:, pl.ds(b * W, W)], inb[t], isem[t])

        def store(k, t):
            b = wid + k * NW
            return pltpu.make_async_copy(
                otb[t], out_hbm.at[pl.ds(b * (W // 2), W // 2), :], osem[t])

        def transpose(t, npg):
            src, dst = inb[t], otb[t]

            @plsc.parallel_loop(0, npg, unroll=4)
            def _(pg):
                for cg in range(P // LANES):
                    h = 1 if cg >= (D // LANES) else 0
                    src_col = 2 * iota + (2 * LANES * pg + h)
                    dst_row = iota + LANES * pg
                    for r in range(LANES):
                        src_row = (LANES * (cg % (D // LANES)) + rot[r]) % D
                        val = plsc.load_gather(src, [src_row, src_col])
                        plsc.store_scatter(
                            dst, [dst_row, LANES * cg + rot[r]], val)

        load(0, 0).start()

        @pl.when(nblk > 1)
        def _():
            load(1, 1).start()

        def pair(kk, carry):
            for t in range(2):
                k = kk * 2 + t

                @pl.when(k < nblk)
                def _():
                    load(k, t).wait()

                    @pl.when(k >= 2)
                    def _():
                        store(k, t).wait()

                    transpose(t, W // 2 // LANES)
                    store(k, t).start()

                    @pl.when(k + 2 < nblk)
                    def _():
                        load(k + 2, t).start()

            return carry

        lax.fori_loop(0, (BASE + 2) // 2, pair, 0)
        for t in range(2):
            @pl.when(nblk > t)
            def _():
                store(0, t).wait()

        if TAIL:
            @pl.when(wid == NW - 1)
            def _():
                pltpu.sync_copy(tail_hbm, otb[0].at[pl.ds(0, TAIL // 2), :])
                pltpu.sync_copy(otb[0].at[pl.ds(0, TAIL // 2), :],
                                out_hbm.at[pl.ds(NBLK * (W // 2), TAIL // 2), :])

    return body(embT, tail)


@functools.partial(jax.jit, static_argnames=("B", "S", "D"))
def _embed(xT, emb2, pos_pad, B, S, D):
    BW = B // NW
    P = 2 * D
    mesh = plsc.VectorSubcoreMesh(core_axis_name="c", subcore_axis_name="s")

    @functools.partial(
        pl.kernel,
        mesh=mesh,
        out_type=jax.ShapeDtypeStruct((S, D, B), jnp.float32),
        scratch_types=[
            pltpu.VMEM((S, BW), jnp.int32),
            pltpu.VMEM((BW,), jnp.int32),
            pltpu.VMEM((BW,), jnp.int32),
            pltpu.VMEM((BW, P), jnp.float32),
            pltpu.VMEM((BW, P), jnp.float32),
            pltpu.VMEM((D, BW), jnp.float32),
            pltpu.VMEM((D, BW), jnp.float32),
            pltpu.VMEM((S, P), jnp.float32),
            pltpu.SemaphoreType.DMA,
            pltpu.SemaphoreType.DMA,
            pltpu.SemaphoreType.DMA,
            pltpu.SemaphoreType.DMA,
        ],
        compiler_params=pltpu.CompilerParams(
            use_tc_tiling_on_sc=True, needs_layout_passes=False),
    )
    def body(xT_hbm, emb_hbm, pos_hbm, out_hbm,
             xloc, idx0, idx1, gb0, gb1, os0, os1, pos_v,
             gsem0, gsem1, osem0, osem1):
        wid = lax.axis_index("s") * NC + lax.axis_index("c")
        b0 = wid * BW
        idx = (idx0, idx1)
        gbuf = (gb0, gb1)
        oslab = (os0, os1)
        gsem = (gsem0, gsem1)
        osem = (osem0, osem1)

        pltpu.sync_copy(pos_hbm, pos_v)
        pltpu.sync_copy(xT_hbm.at[:, pl.ds(b0, BW)], xloc)

        def build_idx(s, t):
            for g in range(BW // LANES):
                sl = pl.ds(g * LANES, LANES)
                idx[t][sl] = lax.shift_right_logical(xloc[s, sl], 1)

        def gather(t):
            return pltpu.make_async_copy(emb_hbm.at[idx[t]], gbuf[t], gsem[t])

        def store(s, t):
            return pltpu.make_async_copy(
                oslab[t], out_hbm.at[s, :, pl.ds(b0, BW)], osem[t])

        def compute(s, t):
            gb, ob = gbuf[t], oslab[t]
            iota = jnp.arange(LANES, dtype=jnp.int32)
            rot = [(iota + r) % LANES for r in range(LANES)]
            for dc in range(D // LANES):
                pchunk = pos_v[s, pl.ds(dc * LANES, LANES)]
                prot = [_rotate(pchunk, rot[r]) for r in range(LANES)]

                @plsc.parallel_loop(0, BW // LANES, unroll=4)
                def _(g):
                    xv = xloc[s, pl.ds(g * LANES, LANES)]
                    colbase = (xv & 1) * D + dc * LANES
                    row_l = iota + g * LANES
                    for r in range(LANES):
                        val = plsc.load_gather(gb, [row_l, colbase + rot[r]])
                        plsc.store_scatter(
                            ob, [dc * LANES + rot[r], row_l], val + prot[r])

        build_idx(0, 0)
        gather(0).start()
        build_idx(1, 1)
        gather(1).start()

        def pair(ss, carry):
            for t in range(2):
                s = ss * 2 + t
                gather(t).wait()

                @pl.when(s >= 2)
                def _():
                    store(s, t).wait()

                compute(s, t)
                store(s, t).start()

                @pl.when(s + 2 < S)
                def _():
                    build_idx(s + 2, t)
                    gather(t).start()

            return carry

        lax.fori_loop(0, S // 2, pair, 0)
        store(S - 2, 0).wait()
        store(S - 1, 1).wait()

    return body(xT, emb2, pos_pad)


def kernel(x, emb_table, pos_table):
    B, S = x.shape
    V, D = emb_table.shape
    xT = jnp.swapaxes(x, 0, 1)
    embT = jnp.swapaxes(emb_table, 0, 1)
    nfull = (V // (2 * D)) * (2 * D)
    tail = emb_table[nfull:].reshape(-1, 2 * D)
    emb2 = _retile(embT, tail, V, D)
    pos_pad = jnp.pad(pos_table, ((0, 0), (0, D)))
    out = _embed(xT, emb2, pos_pad, B, S, D)
    return jnp.transpose(out, (2, 0, 1))


# speedup vs baseline: 1.9962x; 1.9962x over previous
import functools

import jax
import jax.numpy as jnp
from jax import lax
from jax.experimental import pallas as pl
from jax.experimental.pallas import tpu as pltpu
from jax.experimental.pallas import tpu_sc as plsc

NC = 2
NS = 16
NW = NC * NS
LANES = 16

_GATHER_DNUMS = lax.GatherDimensionNumbers(
    offset_dims=(), collapsed_slice_dims=(0,), start_index_map=(0,))


def _rotate(vec, pattern):
    return lax.gather(vec, pattern[:, None], _GATHER_DNUMS, (1,),
                      mode=lax.GatherScatterMode.PROMISE_IN_BOUNDS)


@functools.partial(jax.jit, static_argnames=("V", "D"))
def _retile(embT, tail, V, D):
    P = 2 * D
    W = 4 * D
    NBLK = V // W
    TAIL = V - NBLK * W
    BASE = NBLK // NW
    mesh = plsc.VectorSubcoreMesh(core_axis_name="c", subcore_axis_name="s")

    @functools.partial(
        pl.kernel,
        mesh=mesh,
        out_type=jax.ShapeDtypeStruct((V // 2, P), jnp.float32),
        scratch_types=[
            pltpu.VMEM((D, W), jnp.float32),
            pltpu.VMEM((D, W), jnp.float32),
            pltpu.VMEM((W // 2, P), jnp.float32),
            pltpu.VMEM((W // 2, P), jnp.float32),
            pltpu.SemaphoreType.DMA,
            pltpu.SemaphoreType.DMA,
            pltpu.SemaphoreType.DMA,
            pltpu.SemaphoreType.DMA,
        ],
        compiler_params=pltpu.CompilerParams(
            use_tc_tiling_on_sc=True, needs_layout_passes=False),
    )
    def body(embT_hbm, tail_hbm, out_hbm, in0, in1, ot0, ot1, is0, is1, os0, os1):
        wid = lax.axis_index("s") * NC + lax.axis_index("c")
        inb = (in0, in1)
        otb = (ot0, ot1)
        isem = (is0, is1)
        osem = (os0, os1)
        nblk = BASE + jnp.where(wid < NBLK - BASE * NW, 1, 0)
        iota = jnp.arange(LANES, dtype=jnp.int32)
        rot = [(iota + r) % LANES for r in range(LANES)]

        def load(k, t):
            b = wid + k * NW
            return pltpu.make_async_copy(
                embT_hbm.at[:, pl.ds(b * W, W)], inb[t], isem[t])

        def store(k, t):
            b = wid + k * NW
            return pltpu.make_async_copy(
                otb[t], out_hbm.at[pl.ds(b * (W // 2), W // 2), :], osem[t])

        def transpose(t, npg):
            src, dst = inb[t], otb[t]

            @plsc.parallel_loop(0, npg, unroll=2)
            def _(pg):
                for cg in range(P // LANES):
                    h = 1 if cg >= (D // LANES) else 0
                    src_col = 2 * iota + (2 * LANES * pg + h)
                    dst_row = iota + LANES * pg
                    for r in range(LANES):
                        src_row = (LANES * (cg % (D // LANES)) + rot[r]) % D
                        val = plsc.load_gather(src, [src_row, src_col])
                        plsc.store_scatter(
                            dst, [dst_row, LANES * cg + rot[r]], val)

        load(0, 0).start()

        @pl.when(nblk > 1)
        def _():
            load(1, 1).start()

        def pair(kk, carry):
            for t in range(2):
                k = kk * 2 + t

                @pl.when(k < nblk)
                def _():
                    load(k, t).wait()

                    @pl.when(k >= 2)
                    def _():
                        store(k, t).wait()

                    transpose(t, W // 2 // LANES)
                    store(k, t).start()

                    @pl.when(k + 2 < nblk)
                    def _():
                        load(k + 2, t).start()

            return carry

        lax.fori_loop(0, (BASE + 2) // 2, pair, 0)
        for t in range(2):
            @pl.when(nblk > t)
            def _():
                store(0, t).wait()

        if TAIL:
            @pl.when(wid == NW - 1)
            def _():
                pltpu.sync_copy(tail_hbm, otb[0].at[pl.ds(0, TAIL // 2), :])
                pltpu.sync_copy(otb[0].at[pl.ds(0, TAIL // 2), :],
                                out_hbm.at[pl.ds(NBLK * (W // 2), TAIL // 2), :])

    return body(embT, tail)


@functools.partial(jax.jit, static_argnames=("B", "S", "D"))
def _embed(xT, emb2, pos_pad, B, S, D):
    BW = B // NW
    P = 2 * D
    mesh = plsc.VectorSubcoreMesh(core_axis_name="c", subcore_axis_name="s")

    @functools.partial(
        pl.kernel,
        mesh=mesh,
        out_type=jax.ShapeDtypeStruct((S, D, B), jnp.float32),
        scratch_types=[
            pltpu.VMEM((S, BW), jnp.int32),
            pltpu.VMEM((BW,), jnp.int32),
            pltpu.VMEM((BW,), jnp.int32),
            pltpu.VMEM((BW, P), jnp.float32),
            pltpu.VMEM((BW, P), jnp.float32),
            pltpu.VMEM((D, BW), jnp.float32),
            pltpu.VMEM((D, BW), jnp.float32),
            pltpu.VMEM((S, P), jnp.float32),
            pltpu.SemaphoreType.DMA,
            pltpu.SemaphoreType.DMA,
            pltpu.SemaphoreType.DMA,
            pltpu.SemaphoreType.DMA,
        ],
        compiler_params=pltpu.CompilerParams(
            use_tc_tiling_on_sc=True, needs_layout_passes=False),
    )
    def body(xT_hbm, emb_hbm, pos_hbm, out_hbm,
             xloc, idx0, idx1, gb0, gb1, os0, os1, pos_v,
             gsem0, gsem1, osem0, osem1):
        wid = lax.axis_index("s") * NC + lax.axis_index("c")
        b0 = wid * BW
        idx = (idx0, idx1)
        gbuf = (gb0, gb1)
        oslab = (os0, os1)
        gsem = (gsem0, gsem1)
        osem = (osem0, osem1)

        pltpu.sync_copy(pos_hbm, pos_v)
        pltpu.sync_copy(xT_hbm.at[:, pl.ds(b0, BW)], xloc)

        def build_idx(s, t):
            for g in range(BW // LANES):
                sl = pl.ds(g * LANES, LANES)
                idx[t][sl] = lax.shift_right_logical(xloc[s, sl], 1)

        def gather(t):
            return pltpu.make_async_copy(emb_hbm.at[idx[t]], gbuf[t], gsem[t])

        def store(s, t):
            return pltpu.make_async_copy(
                oslab[t], out_hbm.at[s, :, pl.ds(b0, BW)], osem[t])

        def compute(s, t):
            gb, ob = gbuf[t], oslab[t]
            iota = jnp.arange(LANES, dtype=jnp.int32)
            rot = [(iota + r) % LANES for r in range(LANES)]
            for dc in range(D // LANES):
                pchunk = pos_v[s, pl.ds(dc * LANES, LANES)]
                prot = [_rotate(pchunk, rot[r]) for r in range(LANES)]

                @plsc.parallel_loop(0, BW // LANES, unroll=4)
                def _(g):
                    xv = xloc[s, pl.ds(g * LANES, LANES)]
                    colbase = (xv & 1) * D + dc * LANES
                    row_l = iota + g * LANES
                    for r in range(LANES):
                        val = plsc.load_gather(gb, [row_l, colbase + rot[r]])
                        plsc.store_scatter(
                            ob, [dc * LANES + rot[r], row_l], val + prot[r])

        build_idx(0, 0)
        gather(0).start()
        build_idx(1, 1)
        gather(1).start()

        def pair(ss, carry):
            for t in range(2):
                s = ss * 2 + t
                gather(t).wait()

                @pl.when(s >= 2)
                def _():
                    store(s, t).wait()

                compute(s, t)
                store(s, t).start()

                @pl.when(s + 2 < S)
                def _():
                    build_idx(s + 2, t)
                    gather(t).start()

            return carry

        lax.fori_loop(0, S // 2, pair, 0)
        store(S - 2, 0).wait()
        store(S - 1, 1).wait()

    return body(xT, emb2, pos_pad)


def kernel(x, emb_table, pos_table):
    B, S = x.shape
    V, D = emb_table.shape
    xT = jnp.swapaxes(x, 0, 1)
    embT = jnp.swapaxes(emb_table, 0, 1)
    nfull = (V // (2 * D)) * (2 * D)
    tail = emb_table[nfull:].reshape(-1, 2 * D)
    emb2 = _retile(embT, tail, V, D)
    pos_pad = jnp.pad(pos_table, ((0, 0), (0, D)))
    out = _embed(xT, emb2, pos_pad, B, S, D)
    return jnp.transpose(out, (2, 0, 1))


# speedup vs baseline: 2.3563x; 1.1804x over previous
import functools

import jax
import jax.numpy as jnp
from jax import lax
from jax.experimental import pallas as pl
from jax.experimental.pallas import tpu as pltpu
from jax.experimental.pallas import tpu_sc as plsc

NC = 2
NS = 16
NW = NC * NS
LANES = 16

_GATHER_DNUMS = lax.GatherDimensionNumbers(
    offset_dims=(), collapsed_slice_dims=(0,), start_index_map=(0,))


def _rotate(vec, pattern):
    return lax.gather(vec, pattern[:, None], _GATHER_DNUMS, (1,),
                      mode=lax.GatherScatterMode.PROMISE_IN_BOUNDS)


@functools.partial(jax.jit, static_argnames=("V", "D"))
def _retile(embT, tail, V, D):
    P = 2 * D
    W = 4 * D
    NBLK = V // W
    TAIL = V - NBLK * W
    BASE = NBLK // NW
    mesh = plsc.VectorSubcoreMesh(core_axis_name="c", subcore_axis_name="s")

    @functools.partial(
        pl.kernel,
        mesh=mesh,
        out_type=jax.ShapeDtypeStruct((V // 2, P), jnp.float32),
        scratch_types=[
            pltpu.VMEM((D, W), jnp.float32),
            pltpu.VMEM((D, W), jnp.float32),
            pltpu.VMEM((W // 2, P), jnp.float32),
            pltpu.VMEM((W // 2, P), jnp.float32),
            pltpu.SemaphoreType.DMA,
            pltpu.SemaphoreType.DMA,
            pltpu.SemaphoreType.DMA,
            pltpu.SemaphoreType.DMA,
        ],
        compiler_params=pltpu.CompilerParams(
            use_tc_tiling_on_sc=True, needs_layout_passes=False),
    )
    def body(embT_hbm, tail_hbm, out_hbm, in0, in1, ot0, ot1, is0, is1, os0, os1):
        wid = lax.axis_index("s") * NC + lax.axis_index("c")
        inb = (in0, in1)
        otb = (ot0, ot1)
        isem = (is0, is1)
        osem = (os0, os1)
        nblk = BASE + jnp.where(wid < NBLK - BASE * NW, 1, 0)
        iota = jnp.arange(LANES, dtype=jnp.int32)
        rot = [(iota + r) % LANES for r in range(LANES)]

        def load(k, t):
            b = wid + k * NW
            return pltpu.make_async_copy(
                embT_hbm.at[:, pl.ds(b * W, W)], inb[t], isem[t])

        def store(k, t):
            b = wid + k * NW
            return pltpu.make_async_copy(
                otb[t], out_hbm.at[pl.ds(b * (W // 2), W // 2), :], osem[t])

        def transpose(t, npg):
            src, dst = inb[t], otb[t]

            ncg = P // LANES

            @plsc.parallel_loop(0, npg * ncg, unroll=4)
            def _(i):
                pg = i // ncg
                cg = lax.rem(i, ncg)
                h = jnp.where(cg >= D // LANES, 1, 0).astype(jnp.int32)
                src_col = 2 * iota + (2 * LANES * pg + h)
                dst_row = iota + LANES * pg
                ccg = lax.rem(cg, D // LANES)
                for r in range(LANES):
                    src_row = LANES * ccg + rot[r]
                    val = plsc.load_gather(src, [src_row, src_col])
                    plsc.store_scatter(
                        dst, [dst_row, LANES * cg + rot[r]], val)

        load(0, 0).start()

        @pl.when(nblk > 1)
        def _():
            load(1, 1).start()

        def pair(kk, carry):
            for t in range(2):
                k = kk * 2 + t

                @pl.when(k < nblk)
                def _():
                    load(k, t).wait()

                    @pl.when(k >= 2)
                    def _():
                        store(k, t).wait()

                    transpose(t, W // 2 // LANES)
                    store(k, t).start()

                    @pl.when(k + 2 < nblk)
                    def _():
                        load(k + 2, t).start()

            return carry

        lax.fori_loop(0, (BASE + 2) // 2, pair, 0)
        for t in range(2):
            @pl.when(nblk > t)
            def _():
                store(0, t).wait()

        if TAIL:
            @pl.when(wid == NW - 1)
            def _():
                pltpu.sync_copy(tail_hbm, otb[0].at[pl.ds(0, TAIL // 2), :])
                pltpu.sync_copy(otb[0].at[pl.ds(0, TAIL // 2), :],
                                out_hbm.at[pl.ds(NBLK * (W // 2), TAIL // 2), :])

    return body(embT, tail)


@functools.partial(jax.jit, static_argnames=("B", "S", "D"))
def _embed(xT, emb2, pos_pad, B, S, D):
    BW = B // NW
    P = 2 * D
    mesh = plsc.VectorSubcoreMesh(core_axis_name="c", subcore_axis_name="s")

    @functools.partial(
        pl.kernel,
        mesh=mesh,
        out_type=jax.ShapeDtypeStruct((S, D, B), jnp.float32),
        scratch_types=[
            pltpu.VMEM((S, BW), jnp.int32),
            pltpu.VMEM((BW,), jnp.int32),
            pltpu.VMEM((BW,), jnp.int32),
            pltpu.VMEM((BW, P), jnp.float32),
            pltpu.VMEM((BW, P), jnp.float32),
            pltpu.VMEM((D, BW), jnp.float32),
            pltpu.VMEM((D, BW), jnp.float32),
            pltpu.VMEM((S, P), jnp.float32),
            pltpu.SemaphoreType.DMA,
            pltpu.SemaphoreType.DMA,
            pltpu.SemaphoreType.DMA,
            pltpu.SemaphoreType.DMA,
        ],
        compiler_params=pltpu.CompilerParams(
            use_tc_tiling_on_sc=True, needs_layout_passes=False),
    )
    def body(xT_hbm, emb_hbm, pos_hbm, out_hbm,
             xloc, idx0, idx1, gb0, gb1, os0, os1, pos_v,
             gsem0, gsem1, osem0, osem1):
        wid = lax.axis_index("s") * NC + lax.axis_index("c")
        b0 = wid * BW
        idx = (idx0, idx1)
        gbuf = (gb0, gb1)
        oslab = (os0, os1)
        gsem = (gsem0, gsem1)
        osem = (osem0, osem1)

        pltpu.sync_copy(pos_hbm, pos_v)
        pltpu.sync_copy(xT_hbm.at[:, pl.ds(b0, BW)], xloc)

        def build_idx(s, t):
            for g in range(BW // LANES):
                sl = pl.ds(g * LANES, LANES)
                idx[t][sl] = lax.shift_right_logical(xloc[s, sl], 1)

        def gather(t):
            return pltpu.make_async_copy(emb_hbm.at[idx[t]], gbuf[t], gsem[t])

        def store(s, t):
            return pltpu.make_async_copy(
                oslab[t], out_hbm.at[s, :, pl.ds(b0, BW)], osem[t])

        def compute(s, t):
            gb, ob = gbuf[t], oslab[t]
            iota = jnp.arange(LANES, dtype=jnp.int32)
            rot = [(iota + r) % LANES for r in range(LANES)]
            for dc in range(D // LANES):
                pchunk = pos_v[s, pl.ds(dc * LANES, LANES)]
                prot = [_rotate(pchunk, rot[r]) for r in range(LANES)]

                @plsc.parallel_loop(0, BW // LANES, unroll=4)
                def _(g):
                    xv = xloc[s, pl.ds(g * LANES, LANES)]
                    colbase = (xv & 1) * D + dc * LANES
                    row_l = iota + g * LANES
                    for r in range(LANES):
                        val = plsc.load_gather(gb, [row_l, colbase + rot[r]])
                        plsc.store_scatter(
                            ob, [dc * LANES + rot[r], row_l], val + prot[r])

        build_idx(0, 0)
        gather(0).start()
        build_idx(1, 1)
        gather(1).start()

        def pair(ss, carry):
            for t in range(2):
                s = ss * 2 + t
                gather(t).wait()

                @pl.when(s >= 2)
                def _():
                    store(s, t).wait()

                compute(s, t)
                store(s, t).start()

                @pl.when(s + 2 < S)
                def _():
                    build_idx(s + 2, t)
                    gather(t).start()

            return carry

        lax.fori_loop(0, S // 2, pair, 0)
        store(S - 2, 0).wait()
        store(S - 1, 1).wait()

    return body(xT, emb2, pos_pad)


def kernel(x, emb_table, pos_table):
    B, S = x.shape
    V, D = emb_table.shape
    xT = jnp.swapaxes(x, 0, 1)
    embT = jnp.swapaxes(emb_table, 0, 1)
    nfull = (V // (2 * D)) * (2 * D)
    tail = emb_table[nfull:].reshape(-1, 2 * D)
    emb2 = _retile(embT, tail, V, D)
    pos_pad = jnp.pad(pos_table, ((0, 0), (0, D)))
    out = _embed(xT, emb2, pos_pad, B, S, D)
    return jnp.transpose(out, (2, 0, 1))
